# baseline (speedup 1.0000x reference)
import functools

import jax
import jax.numpy as jnp
from jax import lax
from jax.experimental import pallas as pl
from jax.experimental.pallas import tpu as pltpu
from jax.experimental.pallas import tpu_sc as plsc

_N = 10000
_D = 128
_NPSC = 320
_NSUB = 32
_NPAD = _NPSC * _NSUB
_K = 8
_GC = 16
_G = _GC * _K
_BIG = 1 << 20
_GMAX = (320000 + 7 * _NPAD) // _K


def _seg_max_body(feat_hbm, psrc_hbm, gn_hbm, goff_hbm, out_hbm,
                  idx_v, gnb_v, rows_v, goff_v, acc_v, acc_w, sem0, sem1):
    c = lax.axis_index("c")
    s = lax.axis_index("s")
    w = s * 2 + c
    node_lo = w * _NPSC
    sems = (sem0, sem1)

    pltpu.sync_copy(goff_hbm, goff_v)
    gv = goff_v[pl.ds(w, 16)]
    g_start = gv[0]
    g_end = gv[1]

    neg_inf = jnp.full((16,), -jnp.inf, dtype=jnp.float32)

    def init_row(i, carry):
        for ch in range(_D // 16):
            acc_v[i, pl.ds(ch * 16, 16)] = neg_inf
            acc_w[i, pl.ds(ch * 16, 16)] = neg_inf
        return carry

    lax.fori_loop(0, _NPSC + 1, init_row, 0)

    ga = (g_start // 8) * 8
    nchunks = (g_end - ga + _GC - 1) // _GC

    def issue(cidx, b):
        g0 = ga + cidx * _GC
        pltpu.sync_copy(psrc_hbm.at[pl.ds(g0 * _K, _G)], idx_v.at[b])
        pltpu.sync_copy(gn_hbm.at[pl.ds(g0, _GC)],
                        gnb_v.at[pl.ds(b * 32, _GC)])
        pltpu.async_copy(feat_hbm.at[idx_v.at[b]], rows_v.at[b], sems[b])

    def process(b):
        boff = b * 32
        gl = gnb_v[pl.ds(boff, 16)] - node_lo
        gl = jnp.where((gl < 0) | (gl >= _NPSC), _NPSC, gl)
        gnb_v[pl.ds(boff, 16)] = gl

        def grp2(r2, carry2):
            r = r2 * 2
            d0 = gnb_v[pl.ds(boff + r, 16)][0]
            d1 = gnb_v[pl.ds(boff + r + 1, 16)][0]
            for ch in range(_D // 16):
                sl = pl.ds(ch * 16, 16)
                m0 = rows_v[b, r * _K, sl]
                m1 = rows_v[b, r * _K + _K, sl]
                for k in range(1, _K):
                    m0 = jnp.maximum(m0, rows_v[b, r * _K + k, sl])
                    m1 = jnp.maximum(m1, rows_v[b, r * _K + _K + k, sl])
                acc_v[d0, sl] = jnp.maximum(acc_v[d0, sl], m0)
                acc_w[d1, sl] = jnp.maximum(acc_w[d1, sl], m1)
            return carry2

        lax.fori_loop(0, _GC // 2, grp2, 0)

    @pl.when(nchunks > 0)
    def _prime():
        issue(0, 0)

    def pair(p, carry):
        for b in range(2):
            cidx = p * 2 + b

            @pl.when(cidx < nchunks)
            def _body():
                pltpu.make_async_copy(feat_hbm.at[idx_v.at[b]],
                                      rows_v.at[b], sems[b]).wait()

                @pl.when(cidx + 1 < nchunks)
                def _next():
                    issue(cidx + 1, 1 - b)

                process(b)
        return carry

    lax.fori_loop(0, (nchunks + 1) // 2, pair, 0)

    def merge(i, carry):
        for ch in range(_D // 16):
            sl = pl.ds(ch * 16, 16)
            acc_v[i, sl] = jnp.maximum(acc_v[i, sl], acc_w[i, sl])
        return carry

    lax.fori_loop(0, _NPSC, merge, 0)

    pltpu.sync_copy(acc_v.at[pl.ds(0, _NPSC)],
                    out_hbm.at[pl.ds(node_lo, _NPSC)])


_seg_max = functools.partial(
    pl.kernel,
    out_type=jax.ShapeDtypeStruct((_NPAD, _D), jnp.float32),
    mesh=plsc.VectorSubcoreMesh(core_axis_name="c", subcore_axis_name="s",
                                num_cores=2, num_subcores=16),
    scratch_types=[
        pltpu.VMEM((2, _G), jnp.int32),
        pltpu.VMEM((64,), jnp.int32),
        pltpu.VMEM((2, _G, _D), jnp.float32),
        pltpu.VMEM((48,), jnp.int32),
        pltpu.VMEM((_NPSC + 1, _D), jnp.float32),
        pltpu.VMEM((_NPSC + 1, _D), jnp.float32),
        pltpu.SemaphoreType.DMA,
        pltpu.SemaphoreType.DMA,
    ],
)(_seg_max_body)


def _dense_body(agg_ref, x_ref, wl_ref, wr_ref, b_ref, o_ref, *, relu):
    a = agg_ref[...]
    a = jnp.where(a == -jnp.inf, 0.0, a)
    dn = (((1,), (1,)), ((), ()))
    out = lax.dot_general(a, wl_ref[...], dn, preferred_element_type=jnp.float32)
    out = out + lax.dot_general(x_ref[...], wr_ref[...], dn,
                                preferred_element_type=jnp.float32)
    out = out + b_ref[...][0:1, :]
    nrm = jnp.sqrt(jnp.sum(out * out, axis=1, keepdims=True))
    out = out / jnp.maximum(nrm, 1e-12)
    if relu:
        out = jnp.maximum(out, 0.0)
    o_ref[...] = out


def _dense_layer(agg, x, wl, wr, b, relu):
    m = agg.shape[0]
    tile = 512
    b8 = jnp.broadcast_to(b.reshape(1, _D), (8, _D))
    return pl.pallas_call(
        functools.partial(_dense_body, relu=relu),
        grid=(m // tile,),
        in_specs=[
            pl.BlockSpec((tile, _D), lambda i: (i, 0)),
            pl.BlockSpec((tile, _D), lambda i: (i, 0)),
            pl.BlockSpec((_D, _D), lambda i: (0, 0)),
            pl.BlockSpec((_D, _D), lambda i: (0, 0)),
            pl.BlockSpec((8, _D), lambda i: (0, 0)),
        ],
        out_specs=pl.BlockSpec((tile, _D), lambda i: (i, 0)),
        out_shape=jax.ShapeDtypeStruct((m, _D), jnp.float32),
    )(agg, x, wl, wr, b8)


def kernel(x, edge_index, Wl1, bl1, Wr1, Wl2, bl2, Wr2):
    x = x.astype(jnp.float32)
    ei = edge_index.astype(jnp.int32)
    src, dst = ei[0], ei[1]
    e_cnt = src.shape[0]

    dsts, srcs = lax.sort((dst, src), num_keys=1)
    node_starts = jnp.searchsorted(
        dsts, jnp.arange(_NPAD + 1, dtype=jnp.int32)).astype(jnp.int32)
    deg = node_starts[1:] - node_starts[:-1]
    ngrp = (deg + (_K - 1)) // _K
    gstart = jnp.concatenate(
        [jnp.zeros((1,), jnp.int32), jnp.cumsum(ngrp).astype(jnp.int32)])
    total_g = gstart[_NPAD]
    gidx = jnp.arange(_GMAX, dtype=jnp.int32)
    gn = jnp.searchsorted(gstart, gidx, side='right').astype(jnp.int32) - 1
    gn = jnp.where(gidx < total_g, gn, _BIG)
    gns = jnp.minimum(gn, _NPAD - 1)
    t = (gidx - gstart[gns])[:, None] * _K + jnp.arange(_K, dtype=jnp.int32)
    epos = node_starts[gns][:, None] + t
    valid = (t < deg[gns][:, None]) & (gn[:, None] < _NPAD)
    epos = jnp.where(valid, epos, e_cnt)
    srcs_ext = jnp.concatenate([srcs, jnp.full((1,), _NPAD, jnp.int32)])
    psrc = jnp.take(srcs_ext, epos.reshape(-1))
    psrc = jnp.concatenate([psrc, jnp.zeros((256,), jnp.int32)])
    gn_p = jnp.concatenate([gn, jnp.full((64,), _BIG, jnp.int32)])
    goff = jnp.concatenate(
        [gstart[jnp.arange(33, dtype=jnp.int32) * _NPSC],
         jnp.zeros((15,), jnp.int32)])

    ninf_row = jnp.full((1, _D), -jnp.inf, jnp.float32)
    xp = jnp.concatenate([x, jnp.zeros((_NPAD - _N, _D), jnp.float32)])
    x_t = jnp.concatenate([xp, ninf_row])

    agg1 = _seg_max(x_t, psrc, gn_p, goff)
    h = _dense_layer(agg1, xp, Wl1, Wr1, bl1, relu=True)
    h_t = jnp.concatenate([h, ninf_row])
    agg2 = _seg_max(h_t, psrc, gn_p, goff)
    out = _dense_layer(agg2, h, Wl2, Wr2, bl2, relu=False)
    return out[:_N]


# speedup vs baseline: 7.5438x; 7.5438x over previous
import functools

import jax
import jax.numpy as jnp
from jax import lax
from jax.experimental import pallas as pl
from jax.experimental.pallas import tpu as pltpu
from jax.experimental.pallas import tpu_sc as plsc

_N = 10000
_D = 128
_NPSC = 320
_NSUB = 32
_NPAD = _NPSC * _NSUB
_G = 128
_BIG = 1 << 20


def _seg_max_body(feat_hbm, srcs_hbm, dsts_hbm, starts_hbm, out_hbm,
                  idx_v, dst_v, rows_v, starts_v, acc_v, acc_w, sem0, sem1):
    c = lax.axis_index("c")
    s = lax.axis_index("s")
    w = s * 2 + c
    node_lo = w * _NPSC
    sems = (sem0, sem1)

    pltpu.sync_copy(starts_hbm, starts_v)
    sv = starts_v[pl.ds(w, 16)]
    e_start = sv[0]
    e_end = sv[1]

    neg_inf = jnp.full((16,), -jnp.inf, dtype=jnp.float32)

    def init_row(i, carry):
        for ch in range(_D // 16):
            acc_v[i, pl.ds(ch * 16, 16)] = neg_inf
            acc_w[i, pl.ds(ch * 16, 16)] = neg_inf
        return carry

    lax.fori_loop(0, _NPSC + 1, init_row, 0)

    a_start = (e_start // 8) * 8
    ngroups = (e_end - a_start + _G - 1) // _G

    def issue(g, b):
        e0 = a_start + g * _G
        pltpu.sync_copy(srcs_hbm.at[pl.ds(e0, _G)], idx_v.at[b])
        pltpu.sync_copy(dsts_hbm.at[pl.ds(e0, _G)],
                        dst_v.at[pl.ds(b * (_G + 16), _G)])
        pltpu.async_copy(feat_hbm.at[idx_v.at[b]], rows_v.at[b], sems[b])

    def process(b):
        boff = b * (_G + 16)

        def localize(j, carry2):
            base = boff + j * 16
            dv = dst_v[pl.ds(base, 16)] - node_lo
            dv = jnp.where((dv < 0) | (dv >= _NPSC), _NPSC, dv)
            dst_v[pl.ds(base, 16)] = dv
            return carry2

        lax.fori_loop(0, _G // 16, localize, 0)

        def edge4(i4, carry2):
            i = i4 * 4
            d0 = dst_v[pl.ds(boff + i, 16)][0]
            d1 = dst_v[pl.ds(boff + i + 1, 16)][0]
            d2 = dst_v[pl.ds(boff + i + 2, 16)][0]
            d3 = dst_v[pl.ds(boff + i + 3, 16)][0]
            for ch in range(_D // 16):
                sl = pl.ds(ch * 16, 16)
                acc_v[d0, sl] = jnp.maximum(acc_v[d0, sl], rows_v[b, i, sl])
                acc_w[d1, sl] = jnp.maximum(acc_w[d1, sl],
                                            rows_v[b, i + 1, sl])
                acc_v[d2, sl] = jnp.maximum(acc_v[d2, sl],
                                            rows_v[b, i + 2, sl])
                acc_w[d3, sl] = jnp.maximum(acc_w[d3, sl],
                                            rows_v[b, i + 3, sl])
            return carry2

        lax.fori_loop(0, _G // 4, edge4, 0)

    @pl.when(ngroups > 0)
    def _prime():
        issue(0, 0)

    def pair(p, carry):
        for b in range(2):
            g = p * 2 + b

            @pl.when(g < ngroups)
            def _body():
                pltpu.make_async_copy(feat_hbm.at[idx_v.at[b]],
                                      rows_v.at[b], sems[b]).wait()

                @pl.when(g + 1 < ngroups)
                def _next():
                    issue(g + 1, 1 - b)

                process(b)
        return carry

    lax.fori_loop(0, (ngroups + 1) // 2, pair, 0)

    def merge(i, carry):
        for ch in range(_D // 16):
            sl = pl.ds(ch * 16, 16)
            acc_v[i, sl] = jnp.maximum(acc_v[i, sl], acc_w[i, sl])
        return carry

    lax.fori_loop(0, _NPSC, merge, 0)

    pltpu.sync_copy(acc_v.at[pl.ds(0, _NPSC)],
                    out_hbm.at[pl.ds(node_lo, _NPSC)])


_seg_max = functools.partial(
    pl.kernel,
    out_type=jax.ShapeDtypeStruct((_NPAD, _D), jnp.float32),
    mesh=plsc.VectorSubcoreMesh(core_axis_name="c", subcore_axis_name="s",
                                num_cores=2, num_subcores=16),
    scratch_types=[
        pltpu.VMEM((2, _G), jnp.int32),
        pltpu.VMEM((2 * (_G + 16),), jnp.int32),
        pltpu.VMEM((2, _G, _D), jnp.float32),
        pltpu.VMEM((48,), jnp.int32),
        pltpu.VMEM((_NPSC + 1, _D), jnp.float32),
        pltpu.VMEM((_NPSC + 1, _D), jnp.float32),
        pltpu.SemaphoreType.DMA,
        pltpu.SemaphoreType.DMA,
    ],
)(_seg_max_body)


def _dense_body(agg_ref, x_ref, wl_ref, wr_ref, b_ref, o_ref, *, relu):
    a = agg_ref[...]
    a = jnp.where(a == -jnp.inf, 0.0, a)
    dn = (((1,), (1,)), ((), ()))
    out = lax.dot_general(a, wl_ref[...], dn, preferred_element_type=jnp.float32)
    out = out + lax.dot_general(x_ref[...], wr_ref[...], dn,
                                preferred_element_type=jnp.float32)
    out = out + b_ref[...][0:1, :]
    nrm = jnp.sqrt(jnp.sum(out * out, axis=1, keepdims=True))
    out = out / jnp.maximum(nrm, 1e-12)
    if relu:
        out = jnp.maximum(out, 0.0)
    o_ref[...] = out


def _dense_layer(agg, x, wl, wr, b, relu):
    m = agg.shape[0]
    tile = 512
    b8 = jnp.broadcast_to(b.reshape(1, _D), (8, _D))
    return pl.pallas_call(
        functools.partial(_dense_body, relu=relu),
        grid=(m // tile,),
        in_specs=[
            pl.BlockSpec((tile, _D), lambda i: (i, 0)),
            pl.BlockSpec((tile, _D), lambda i: (i, 0)),
            pl.BlockSpec((_D, _D), lambda i: (0, 0)),
            pl.BlockSpec((_D, _D), lambda i: (0, 0)),
            pl.BlockSpec((8, _D), lambda i: (0, 0)),
        ],
        out_specs=pl.BlockSpec((tile, _D), lambda i: (i, 0)),
        out_shape=jax.ShapeDtypeStruct((m, _D), jnp.float32),
    )(agg, x, wl, wr, b8)


def kernel(x, edge_index, Wl1, bl1, Wr1, Wl2, bl2, Wr2):
    x = x.astype(jnp.float32)
    ei = edge_index.astype(jnp.int32)
    src, dst = ei[0], ei[1]

    dsts, srcs = lax.sort((dst, src), num_keys=1)
    bounds = jnp.arange(33, dtype=jnp.int32) * _NPSC
    starts = jnp.searchsorted(dsts, bounds).astype(jnp.int32)
    starts = jnp.concatenate([starts, jnp.zeros((15,), jnp.int32)])
    srcs_p = jnp.concatenate([srcs, jnp.zeros((2 * _G,), jnp.int32)])
    dsts_p = jnp.concatenate([dsts, jnp.full((2 * _G,), _BIG, jnp.int32)])

    xp = jnp.concatenate([x, jnp.zeros((_NPAD - _N, _D), jnp.float32)])

    agg1 = _seg_max(x, srcs_p, dsts_p, starts)
    h = _dense_layer(agg1, xp, Wl1, Wr1, bl1, relu=True)
    agg2 = _seg_max(h, srcs_p, dsts_p, starts)
    out = _dense_layer(agg2, h, Wl2, Wr2, bl2, relu=False)
    return out[:_N]
